# baseline (speedup 1.0000x reference)
import functools

import jax
import jax.numpy as jnp
from jax import lax
from jax.experimental import pallas as pl
from jax.experimental.pallas import tpu as pltpu
from jax.experimental.pallas import tpu_sc as plsc

B = 16384
S = 10
F = 32
NC = 2
NS = 16
NW = NC * NS
BPW = B // NW
CB = 256
NCHUNK = BPW // CB
CN = CB * S
GB = 8
GN = GB * S
NVEC = GN // 16

_mesh = plsc.VectorSubcoreMesh(core_axis_name="c", subcore_axis_name="s")


def _hsum_all_lanes(p, lane):
    for sft in (8, 4, 2, 1):
        p = p + jnp.take_along_axis(p, jnp.bitwise_xor(lane, sft), axis=0)
    return p


@functools.partial(
    pl.kernel,
    mesh=_mesh,
    compiler_params=pltpu.CompilerParams(use_tc_tiling_on_sc=False),
    out_type=jax.ShapeDtypeStruct((B * S,), jnp.float32),
    scratch_types=[
        pltpu.VMEM((CN,), jnp.int32),
        pltpu.VMEM((CB,), jnp.int32),
        pltpu.VMEM((CN, F), jnp.float32),
        pltpu.VMEM((CB, F), jnp.float32),
        pltpu.VMEM((CN,), jnp.float32),
        pltpu.SemaphoreType.DMA,
    ],
)
def _score_kernel(item_hbm, user_hbm, doc_hbm, uemb_hbm, out_hbm,
                  iidx_v, uidx_v, doc_v, usr_v, sc_v, sem):
    wid = lax.axis_index("c") * NS + lax.axis_index("s")
    for chunk in range(NCHUNK):
        nbase = wid * BPW * S + chunk * CN
        bbase = wid * BPW + chunk * CB
        pltpu.sync_copy(item_hbm.at[pl.ds(nbase, CN)], iidx_v)
        pltpu.sync_copy(user_hbm.at[pl.ds(bbase, CB)], uidx_v)
        cp_doc = pltpu.async_copy(doc_hbm.at[iidx_v], doc_v, sem)
        cp_usr = pltpu.async_copy(uemb_hbm.at[uidx_v], usr_v, sem)
        cp_doc.wait()
        cp_usr.wait()

        def body(g, carry):
            base_b = g * GB
            base_n = g * GN
            lane = lax.iota(jnp.int32, 16)
            accs = [jnp.zeros((16,), jnp.float32)] * NVEC
            for i2 in range(GB):
                u0 = usr_v[base_b + i2, pl.ds(0, 16)]
                u1 = usr_v[base_b + i2, pl.ds(16, 16)]
                for s in range(S):
                    n2 = i2 * S + s
                    d0 = doc_v[base_n + n2, pl.ds(0, 16)]
                    d1 = doc_v[base_n + n2, pl.ds(16, 16)]
                    tot = _hsum_all_lanes(d0 * u0 + d1 * u1, lane)
                    v, ln = divmod(n2, 16)
                    accs[v] = jnp.where(lane == ln, tot, accs[v])
            for v in range(NVEC):
                sc_v[pl.ds(base_n + v * 16, 16)] = accs[v]
            return carry

        lax.fori_loop(0, CB // GB, body, 0)
        pltpu.sync_copy(sc_v, out_hbm.at[pl.ds(nbase, CN)])


def kernel(item_ids, user_ids, docEmbed, userEmbed):
    flat_items = item_ids.reshape(-1).astype(jnp.int32)
    uids = user_ids.astype(jnp.int32)
    out = _score_kernel(flat_items, uids, docEmbed, userEmbed)
    return out.reshape(B, S)
